# Initial kernel scaffold; baseline (speedup 1.0000x reference)
import jax
import jax.numpy as jnp
from jax.experimental import pallas as pl


def kernel(x, weight, a):
    raise NotImplementedError("write your pallas kernel here")


# speedup vs baseline: 14.4750x; 14.4750x over previous
import functools

import jax
import jax.numpy as jnp
from jax.experimental import pallas as pl
from jax.experimental.pallas import tpu as pltpu

N = 8192
K = 32
D = 7
BR = 256
BISECT_ITERS = 30


def _body(xf_ref, xft_ref, w_ref, a_ref, q_ref, y_ref):
    a = a_ref[...]
    fw = jax.nn.softmax(a, axis=1)
    xf = xf_ref[...]
    out = jnp.dot(xf, w_ref[...].T, preferred_element_type=jnp.float32)
    out = jnp.clip(out * fw, -1.0, 1.0)

    kt = xft_ref[...]
    kn2 = jnp.sum(kt * kt, axis=0, keepdims=True)
    kt_n = kt * jax.lax.rsqrt(kn2)

    q = q_ref[...]
    qn2 = jnp.sum(q * q, axis=1, keepdims=True)
    q_n = q * jax.lax.rsqrt(qn2)

    s = jax.lax.dot_general(
        q_n, kt_n, (((1,), (0,)), ((), ())),
        preferred_element_type=jnp.float32,
    )

    kf = jnp.float32(K)

    def bisect(_, carry):
        lo, hi = carry
        mid = 0.5 * (lo + hi)
        cnt = jnp.sum((s >= mid).astype(jnp.float32), axis=1, keepdims=True)
        ge = cnt >= kf
        return jnp.where(ge, mid, lo), jnp.where(ge, hi, mid)

    lo0 = jnp.full((BR, 1), -1.1, jnp.float32)
    hi0 = jnp.full((BR, 1), 1.1, jnp.float32)
    lo, _ = jax.lax.fori_loop(0, BISECT_ITERS, bisect, (lo0, hi0))

    w = jnp.where(s >= lo, jnp.exp(s), 0.0)
    z = jnp.sum(w, axis=1, keepdims=True)
    y = jnp.dot(w, out, preferred_element_type=jnp.float32)
    y_ref[...] = y / z


@jax.jit
def kernel(x, weight, a):
    xf = x[:, :D]
    xft = xf.T
    a2 = a.reshape(1, D)
    grid = N // BR
    y = pl.pallas_call(
        _body,
        grid=(grid,),
        in_specs=[
            pl.BlockSpec((N, D), lambda i: (0, 0)),
            pl.BlockSpec((D, N), lambda i: (0, 0)),
            pl.BlockSpec((D, D), lambda i: (0, 0)),
            pl.BlockSpec((1, D), lambda i: (0, 0)),
            pl.BlockSpec((BR, D), lambda i: (i, 0)),
        ],
        out_specs=pl.BlockSpec((BR, D), lambda i: (i, 0)),
        out_shape=jax.ShapeDtypeStruct((N, D), jnp.float32),
        compiler_params=pltpu.CompilerParams(
            dimension_semantics=("arbitrary",),
        ),
    )(xf, xft, weight, a2, xf)
    return y
